# Initial kernel scaffold; baseline (speedup 1.0000x reference)
import functools

import jax
import jax.numpy as jnp
import numpy as np
from jax.experimental import pallas as pl
from jax.experimental.pallas import tpu as pltpu

NUM_HEADS = 4
FILTERS = 256
NUM_CLASSES = 80
NUM_ANCHORS = 9
TOPK = 200
STRIDES = (8, 16, 32)
B = 2
HW_LIST = ((28, 28), (14, 14), (7, 7))


def _make_anchor_boxes():
    all_a = []
    for (h, w), s in zip(HW_LIST, STRIDES):
        scales = np.array([2.0 ** 0, 2.0 ** (1.0 / 3.0), 2.0 ** (2.0 / 3.0)]) * 4.0 * s
        ratios = np.array([0.5, 1.0, 2.0])
        ws = (scales[None, :] * np.sqrt(1.0 / ratios)[:, None]).reshape(-1)
        hs = (scales[None, :] * np.sqrt(ratios)[:, None]).reshape(-1)
        cx = (np.arange(w) + 0.5) * s
        cy = (np.arange(h) + 0.5) * s
        cyg, cxg = np.meshgrid(cy, cx, indexing='ij')
        centers = np.stack([cxg, cyg], -1).reshape(-1, 1, 2)
        wh = np.stack([ws, hs], -1)[None]
        boxes = np.concatenate([centers - wh / 2.0, centers + wh / 2.0], -1)
        all_a.append(boxes.reshape(-1, 4))
    return np.concatenate(all_a, 0).astype(np.float32)


def _valid_mask(H, W):
    m = np.zeros((H + 2, W + 2), np.float32)
    m[1:H + 1, 1:W + 1] = 1.0
    return np.tile(m.reshape(-1), B)[:, None]


def _tower_kernel(x_ref, w_ref, b_ref, wf_ref, bf_ref, mask_ref, out_ref,
                  a_ref, c_ref, *, H, W, rows, lo, hi):
    M = hi - lo
    offs = [(dy - 1) * (W + 2) + (dx - 1) for dy in range(3) for dx in range(3)]

    def conv(src_ref, wget, nout):
        acc = jnp.zeros((M, nout), jnp.float32)
        for t, off in enumerate(offs):
            a = src_ref[pl.ds(lo + off, M), :]
            acc = acc + jax.lax.dot_general(
                a, wget(t), (((1,), (0,)), ((), ())),
                preferred_element_type=jnp.float32,
                precision=jax.lax.Precision.HIGHEST)
        return acc

    a_ref[pl.ds(0, lo), :] = jnp.zeros((lo, FILTERS), jnp.float32)
    a_ref[pl.ds(hi, rows - hi), :] = jnp.zeros((rows - hi, FILTERS), jnp.float32)
    c_ref[pl.ds(0, lo), :] = jnp.zeros((lo, FILTERS), jnp.float32)
    c_ref[pl.ds(hi, rows - hi), :] = jnp.zeros((rows - hi, FILTERS), jnp.float32)

    mask = mask_ref[pl.ds(lo, M), :]
    srcs = [x_ref, a_ref, c_ref, a_ref, c_ref]
    for i in range(NUM_HEADS):
        acc = conv(srcs[i], lambda t, i=i: w_ref[i, t, :, :], FILTERS)
        acc = jnp.maximum(acc + b_ref[i:i + 1, :], 0.0) * mask
        srcs[i + 1][pl.ds(lo, M), :] = acc

    nout = out_ref.shape[1]
    acc = conv(srcs[NUM_HEADS], lambda t: wf_ref[t, :, :], nout)
    out_ref[pl.ds(lo, M), :] = acc + bf_ref[0:1, :]


@functools.partial(jax.jit, static_argnames=("H", "W", "cout"))
def _run_tower(xp_flat, w9, b, wf9, bf, mask, *, H, W, cout):
    rows = B * (H + 2) * (W + 2)
    lo = W + 3
    hi = rows - (W + 3)
    f = functools.partial(_tower_kernel, H=H, W=W, rows=rows, lo=lo, hi=hi)
    return pl.pallas_call(
        f,
        out_shape=jax.ShapeDtypeStruct((rows, cout), jnp.float32),
        scratch_shapes=[pltpu.VMEM((rows, FILTERS), jnp.float32),
                        pltpu.VMEM((rows, FILTERS), jnp.float32)],
    )(xp_flat, w9, b, wf9, bf, mask)


def _decode_boxes(deltas, anchors):
    aw = anchors[..., 2] - anchors[..., 0]
    ah = anchors[..., 3] - anchors[..., 1]
    ax = anchors[..., 0] + aw / 2.0
    ay = anchors[..., 1] + ah / 2.0
    dx, dy, dw, dh = deltas[..., 0], deltas[..., 1], deltas[..., 2], deltas[..., 3]
    px = dx * aw + ax
    py = dy * ah + ay
    pw = jnp.exp(jnp.clip(dw, -10.0, 4.0)) * aw
    ph = jnp.exp(jnp.clip(dh, -10.0, 4.0)) * ah
    return jnp.stack([px - pw / 2.0, py - ph / 2.0, px + pw / 2.0, py + ph / 2.0], -1)


def kernel(feat_p3, feat_p4, feat_p5, cls_w, cls_b, cls_wf, cls_bf,
           box_w, box_b, box_wf, box_bf):
    feats = [feat_p3, feat_p4, feat_p5]
    cls_w9 = cls_w.reshape(NUM_HEADS, 9, FILTERS, FILTERS)
    box_w9 = box_w.reshape(NUM_HEADS, 9, FILTERS, FILTERS)
    cls_wf9 = cls_wf.reshape(9, FILTERS, NUM_ANCHORS * NUM_CLASSES)
    box_wf9 = box_wf.reshape(9, FILTERS, 4 * NUM_ANCHORS)
    cls_bf2 = cls_bf.reshape(1, -1)
    box_bf2 = box_bf.reshape(1, -1)

    cls_outs, box_outs = [], []
    for x, (H, W) in zip(feats, HW_LIST):
        xp = jnp.pad(x, ((0, 0), (1, 1), (1, 1), (0, 0))).reshape(-1, FILTERS)
        mask = jnp.asarray(_valid_mask(H, W))
        c = _run_tower(xp, cls_w9, cls_b, cls_wf9, cls_bf2, mask,
                       H=H, W=W, cout=NUM_ANCHORS * NUM_CLASSES)
        bb = _run_tower(xp, box_w9, box_b, box_wf9, box_bf2, mask,
                        H=H, W=W, cout=4 * NUM_ANCHORS)
        c = c.reshape(B, H + 2, W + 2, -1)[:, 1:H + 1, 1:W + 1, :]
        bb = bb.reshape(B, H + 2, W + 2, -1)[:, 1:H + 1, 1:W + 1, :]
        cls_outs.append(c.reshape(B, -1, NUM_CLASSES))
        box_outs.append(bb.reshape(B, -1, 4))

    cls = jnp.concatenate(cls_outs, 1)
    deltas = jnp.concatenate(box_outs, 1)
    anchors = jnp.asarray(_make_anchor_boxes())
    boxes = _decode_boxes(deltas, anchors[None])
    scores = jnp.max(jax.nn.sigmoid(cls), -1)
    topv, topi = jax.lax.top_k(scores, TOPK)
    topb = jnp.take_along_axis(boxes, topi[:, :, None], axis=1)
    return jnp.concatenate([topb, topv[:, :, None]], -1)


# speedup vs baseline: 1.0435x; 1.0435x over previous
import functools

import jax
import jax.numpy as jnp
import numpy as np
from jax.experimental import pallas as pl
from jax.experimental.pallas import tpu as pltpu

NUM_HEADS = 4
FILTERS = 256
NUM_CLASSES = 80
NUM_ANCHORS = 9
TOPK = 200
STRIDES = (8, 16, 32)
B = 2
HW_LIST = ((28, 28), (14, 14), (7, 7))


def _make_anchor_boxes():
    all_a = []
    for (h, w), s in zip(HW_LIST, STRIDES):
        scales = np.array([2.0 ** 0, 2.0 ** (1.0 / 3.0), 2.0 ** (2.0 / 3.0)]) * 4.0 * s
        ratios = np.array([0.5, 1.0, 2.0])
        ws = (scales[None, :] * np.sqrt(1.0 / ratios)[:, None]).reshape(-1)
        hs = (scales[None, :] * np.sqrt(ratios)[:, None]).reshape(-1)
        cx = (np.arange(w) + 0.5) * s
        cy = (np.arange(h) + 0.5) * s
        cyg, cxg = np.meshgrid(cy, cx, indexing='ij')
        centers = np.stack([cxg, cyg], -1).reshape(-1, 1, 2)
        wh = np.stack([ws, hs], -1)[None]
        boxes = np.concatenate([centers - wh / 2.0, centers + wh / 2.0], -1)
        all_a.append(boxes.reshape(-1, 4))
    return np.concatenate(all_a, 0).astype(np.float32)


def _valid_mask(H, W):
    m = np.zeros((H + 2, W + 2), np.float32)
    m[1:H + 1, 1:W + 1] = 1.0
    return np.tile(m.reshape(-1), B)[:, None]


def _tower_kernel(x_ref, w_ref, b_ref, wf_ref, bf_ref, mask_ref, out_ref,
                  x_bf, p_ref, q_ref, acc_ref, *, H, W, rows, lo, hi):
    M = hi - lo
    offs = [(dy - 1) * (W + 2) + (dx - 1) for dy in range(3) for dx in range(3)]

    def conv(src_ref, wget, dst_ref):
        for t, off in enumerate(offs):
            a = src_ref[pl.ds(lo + off, M), :]
            d = jax.lax.dot_general(
                a, wget(t), (((1,), (0,)), ((), ())),
                preferred_element_type=jnp.float32)
            if t == 0:
                dst_ref[pl.ds(lo, M), :] = d
            else:
                dst_ref[pl.ds(lo, M), :] += d

    x_bf[...] = x_ref[...].astype(jnp.bfloat16)
    zb = jnp.zeros((lo, FILTERS), jnp.bfloat16)
    ze = jnp.zeros((rows - hi, FILTERS), jnp.bfloat16)
    p_ref[pl.ds(0, lo), :] = zb
    p_ref[pl.ds(hi, rows - hi), :] = ze
    q_ref[pl.ds(0, lo), :] = zb
    q_ref[pl.ds(hi, rows - hi), :] = ze

    mask = mask_ref[pl.ds(lo, M), :]
    srcs = [x_bf, p_ref, q_ref, p_ref, q_ref]
    for i in range(NUM_HEADS):
        conv(srcs[i], lambda t, i=i: w_ref[i, t, :, :], acc_ref)
        srcs[i + 1][pl.ds(lo, M), :] = (
            jnp.maximum(acc_ref[pl.ds(lo, M), :] + b_ref[i:i + 1, :], 0.0)
            * mask).astype(jnp.bfloat16)

    conv(srcs[NUM_HEADS], lambda t: wf_ref[t, :, :], out_ref)
    out_ref[pl.ds(lo, M), :] = out_ref[pl.ds(lo, M), :] + bf_ref[0:1, :]


@functools.partial(jax.jit, static_argnames=("H", "W", "cout"))
def _run_tower(xp_flat, w9, b, wf9, bf, mask, *, H, W, cout):
    rows = B * (H + 2) * (W + 2)
    lo = W + 3
    hi = rows - (W + 3)
    f = functools.partial(_tower_kernel, H=H, W=W, rows=rows, lo=lo, hi=hi)
    return pl.pallas_call(
        f,
        out_shape=jax.ShapeDtypeStruct((rows, cout), jnp.float32),
        scratch_shapes=[pltpu.VMEM((rows, FILTERS), jnp.bfloat16),
                        pltpu.VMEM((rows, FILTERS), jnp.bfloat16),
                        pltpu.VMEM((rows, FILTERS), jnp.bfloat16),
                        pltpu.VMEM((rows, FILTERS), jnp.float32)],
    )(xp_flat, w9, b, wf9, bf, mask)


def _decode_boxes(deltas, anchors):
    aw = anchors[..., 2] - anchors[..., 0]
    ah = anchors[..., 3] - anchors[..., 1]
    ax = anchors[..., 0] + aw / 2.0
    ay = anchors[..., 1] + ah / 2.0
    dx, dy, dw, dh = deltas[..., 0], deltas[..., 1], deltas[..., 2], deltas[..., 3]
    px = dx * aw + ax
    py = dy * ah + ay
    pw = jnp.exp(jnp.clip(dw, -10.0, 4.0)) * aw
    ph = jnp.exp(jnp.clip(dh, -10.0, 4.0)) * ah
    return jnp.stack([px - pw / 2.0, py - ph / 2.0, px + pw / 2.0, py + ph / 2.0], -1)


def kernel(feat_p3, feat_p4, feat_p5, cls_w, cls_b, cls_wf, cls_bf,
           box_w, box_b, box_wf, box_bf):
    feats = [feat_p3, feat_p4, feat_p5]
    cls_w9 = cls_w.reshape(NUM_HEADS, 9, FILTERS, FILTERS).astype(jnp.bfloat16)
    box_w9 = box_w.reshape(NUM_HEADS, 9, FILTERS, FILTERS).astype(jnp.bfloat16)
    cls_wf9 = cls_wf.reshape(9, FILTERS, NUM_ANCHORS * NUM_CLASSES).astype(jnp.bfloat16)
    box_wf9 = box_wf.reshape(9, FILTERS, 4 * NUM_ANCHORS).astype(jnp.bfloat16)
    cls_bf2 = cls_bf.reshape(1, -1)
    box_bf2 = box_bf.reshape(1, -1)

    cls_outs, box_outs = [], []
    for x, (H, W) in zip(feats, HW_LIST):
        xp = jnp.pad(x, ((0, 0), (1, 1), (1, 1), (0, 0))).reshape(-1, FILTERS)
        mask = jnp.asarray(_valid_mask(H, W))
        c = _run_tower(xp, cls_w9, cls_b, cls_wf9, cls_bf2, mask,
                       H=H, W=W, cout=NUM_ANCHORS * NUM_CLASSES)
        bb = _run_tower(xp, box_w9, box_b, box_wf9, box_bf2, mask,
                        H=H, W=W, cout=4 * NUM_ANCHORS)
        c = c.reshape(B, H + 2, W + 2, -1)[:, 1:H + 1, 1:W + 1, :]
        bb = bb.reshape(B, H + 2, W + 2, -1)[:, 1:H + 1, 1:W + 1, :]
        cls_outs.append(c.reshape(B, -1, NUM_CLASSES))
        box_outs.append(bb.reshape(B, -1, 4))

    cls = jnp.concatenate(cls_outs, 1)
    deltas = jnp.concatenate(box_outs, 1)
    anchors = jnp.asarray(_make_anchor_boxes())
    boxes = _decode_boxes(deltas, anchors[None])
    scores = jnp.max(jax.nn.sigmoid(cls), -1)
    topv, topi = jax.lax.top_k(scores, TOPK)
    topb = jnp.take_along_axis(boxes, topi[:, :, None], axis=1)
    return jnp.concatenate([topb, topv[:, :, None]], -1)
